# baseline (speedup 1.0000x reference)
import functools

import jax
import jax.numpy as jnp
from jax import lax
from jax.experimental import pallas as pl
from jax.experimental.pallas import tpu as pltpu
from jax.experimental.pallas import tpu_sc as plsc


def _gather_rows_sc(input_ids, emb_table):
    B = input_ids.shape[0]
    V, H = emb_table.shape
    info = plsc.get_sparse_core_info()
    nw = info.num_cores * info.num_subcores
    b_per_w = B // nw

    mesh = plsc.VectorSubcoreMesh(core_axis_name="c", subcore_axis_name="s")

    @functools.partial(
        pl.kernel,
        mesh=mesh,
        out_type=jax.ShapeDtypeStruct((B, H), jnp.float32),
        compiler_params=pltpu.CompilerParams(use_tc_tiling_on_sc=False),
        scratch_types=[
            pltpu.VMEM((b_per_w,), jnp.int32),
            pltpu.VMEM((b_per_w, H), jnp.float32),
            pltpu.SemaphoreType.DMA,
        ],
    )
    def gather_k(idx_hbm, table_hbm, out_hbm, idx_v, rows_v, sem):
        wid = lax.axis_index("s") * info.num_cores + lax.axis_index("c")
        base = wid * b_per_w
        pltpu.sync_copy(idx_hbm.at[pl.ds(base, b_per_w)], idx_v)
        pltpu.async_copy(table_hbm.at[idx_v], rows_v, sem).wait()
        pltpu.sync_copy(rows_v, out_hbm.at[pl.ds(base, b_per_w)])

    return gather_k(input_ids, emb_table)


def _project_tc_t(x, W, b, block_v=2048, ring=6):
    B, H = x.shape
    V = W.shape[0]
    nv = pl.cdiv(V, block_v)
    last = nv - 1
    tail = V - last * block_v

    def mm_k(w_ref, x_ref, b_ref, o_hbm, o_vmem, sems):
        i = pl.program_id(0)
        s = lax.rem(i, ring)

        for t in range(ring):
            @pl.when((s == t) & (i >= ring))
            def _wait_slot(t=t):
                pltpu.make_async_copy(
                    o_vmem.at[t], o_hbm.at[pl.ds(0, block_v), :], sems.at[t]
                ).wait()

        o_vmem[s, :, :] = (
            lax.dot_general(
                w_ref[...].astype(jnp.bfloat16),
                x_ref[...].astype(jnp.bfloat16),
                (((1,), (1,)), ((), ())),
                preferred_element_type=jnp.float32,
            )
            + b_ref[...]
        )

        for t in range(ring):
            @pl.when((s == t) & (i < last))
            def _copy_full(t=t):
                pltpu.make_async_copy(
                    o_vmem.at[t], o_hbm.at[pl.ds(i * block_v, block_v), :],
                    sems.at[t],
                ).start()

        ls = last % ring

        @pl.when(i == last)
        def _copy_tail_and_drain():
            pltpu.make_async_copy(
                o_vmem.at[ls, pl.ds(0, tail), :],
                o_hbm.at[pl.ds(last * block_v, tail), :],
                sems.at[ls],
            ).start()
            for k in range(min(ring, nv)):
                step = last - k
                t = step % ring
                if step == last:
                    pltpu.make_async_copy(
                        o_vmem.at[t, pl.ds(0, tail), :],
                        o_hbm.at[pl.ds(last * block_v, tail), :],
                        sems.at[t],
                    ).wait()
                else:
                    pltpu.make_async_copy(
                        o_vmem.at[t], o_hbm.at[pl.ds(0, block_v), :],
                        sems.at[t],
                    ).wait()

    return pl.pallas_call(
        mm_k,
        grid=(nv,),
        in_specs=[
            pl.BlockSpec((block_v, H), lambda i: (i, 0)),
            pl.BlockSpec((B, H), lambda i: (0, 0)),
            pl.BlockSpec((block_v, 1), lambda i: (i, 0)),
        ],
        out_specs=pl.BlockSpec(memory_space=pl.ANY),
        out_shape=jax.ShapeDtypeStruct((V, B), jnp.float32),
        scratch_shapes=[
            pltpu.VMEM((ring, block_v, B), jnp.float32),
            pltpu.SemaphoreType.DMA((ring,)),
        ],
    )(W, x, b.reshape(V, 1))


def kernel(input_ids, emb_table, W, b):
    x = _gather_rows_sc(input_ids.astype(jnp.int32), emb_table)
    out_t = _project_tc_t(x, W, b)
    return out_t.T
